# baseline (device time: 22261 ns/iter reference)
import os

import jax
import jax.numpy as jnp
from jax import lax
from jax.experimental import pallas as pl
from jax.experimental.pallas import tpu as pltpu

P = 32
EPS = 1e-5
LANES = 128
M = 2048
ROWS = M // LANES
TILE = 2 * ROWS
IN_CHUNKS = 8
IN_R = M // IN_CHUNKS
PACK_R = IN_R // LANES
OUT_CHUNKS = 4
OUT_R = M // OUT_CHUNKS

_OFFSETS = sorted(range(1, P), key=lambda j: min(j, P - j))


def kernel(x, gamma, beta):
    m, n_loc = x.shape
    n_glob = float(n_loc * P)

    gamma2d = gamma.reshape(1, n_loc)
    beta2d = beta.reshape(1, n_loc)

    def body(x_hbm, g_ref, b_ref, o_hbm, xv_ref, gx_ref, comm_ref, obuf_ref,
             in_sems, out_sems, send_sems, recv_sems):
        my = lax.axis_index("i")

        barrier = pltpu.get_barrier_semaphore()
        for j in _OFFSETS:
            pl.semaphore_signal(
                barrier,
                inc=1,
                device_id=((my + j) % P,),
                device_id_type=pl.DeviceIdType.MESH,
            )

        in_dmas = []
        for c in range(IN_CHUNKS):
            dma = pltpu.make_async_copy(
                x_hbm.at[pl.ds(c * IN_R, IN_R), :],
                xv_ref.at[pl.ds(c * IN_R, IN_R), :],
                in_sems.at[c],
            )
            dma.start()
            in_dmas.append(dma)

        gb = g_ref[:, :].astype(jnp.bfloat16)
        for c in range(IN_CHUNKS):
            in_dmas[c].wait()
            xc = xv_ref[pl.ds(c * IN_R, IN_R), :]
            r0 = c * PACK_R
            comm_ref[0, r0:r0 + PACK_R, :] = (
                jnp.sum(xc, axis=1).reshape(PACK_R, LANES).astype(jnp.bfloat16)
            )
            comm_ref[0, ROWS + r0:ROWS + r0 + PACK_R, :] = (
                jnp.sum(xc * xc, axis=1)
                .reshape(PACK_R, LANES)
                .astype(jnp.bfloat16)
            )
            gx_ref[pl.ds(c * IN_R, IN_R), :] = gb * xc.astype(jnp.bfloat16)

        pl.semaphore_wait(barrier, P - 1)

        rdmas = []
        if not os.environ.get("ABLATE_COMM"):
            for j in _OFFSETS:
                slot = P - j
                rdma = pltpu.make_async_remote_copy(
                    src_ref=comm_ref.at[0],
                    dst_ref=comm_ref.at[slot],
                    send_sem=send_sems.at[j],
                    recv_sem=recv_sems.at[slot],
                    device_id=((my + j) % P,),
                    device_id_type=pl.DeviceIdType.MESH,
                )
                rdma.start()
                rdmas.append(rdma)
        for rdma in rdmas:
            rdma.wait()

        total = jnp.sum(comm_ref[:, :, :].astype(jnp.float32), axis=0)

        row_ids = lax.broadcasted_iota(jnp.int32, (m, ROWS), 0)
        sel = (lax.broadcasted_iota(jnp.int32, (m, ROWS), 1)
               == row_ids // LANES).astype(jnp.float32)
        lane_mask = (
            lax.broadcasted_iota(jnp.int32, (m, LANES), 1)
            == lax.broadcasted_iota(jnp.int32, (m, LANES), 0) % LANES
        )

        def unpack(packed):
            spread = jnp.dot(sel, packed, preferred_element_type=jnp.float32)
            return jnp.sum(
                jnp.where(lane_mask, spread, 0.0), axis=1, keepdims=True
            )

        mean = unpack(total[0:ROWS, :]) / n_glob
        var = unpack(total[ROWS:, :]) / n_glob - mean * mean
        rstd = lax.rsqrt(var + EPS)

        r_col = rstd.astype(jnp.bfloat16)
        mr_col = (mean * rstd).astype(jnp.bfloat16)
        bb = b_ref[:, :].astype(jnp.bfloat16)
        out_dmas = []
        for c in range(OUT_CHUNKS):
            rows = pl.ds(c * OUT_R, OUT_R)
            obuf_ref[c, :, :] = (
                gx_ref[rows, :] * r_col[c * OUT_R:(c + 1) * OUT_R, :]
                + (bb - gb * mr_col[c * OUT_R:(c + 1) * OUT_R, :])
            )
            dma = pltpu.make_async_copy(
                obuf_ref.at[c], o_hbm.at[rows, :], out_sems.at[c]
            )
            dma.start()
            out_dmas.append(dma)
        for dma in out_dmas:
            dma.wait()

    return pl.pallas_call(
        body,
        out_shape=jax.ShapeDtypeStruct((m, n_loc), jnp.bfloat16),
        in_specs=[
            pl.BlockSpec(memory_space=pl.ANY),
            pl.BlockSpec(memory_space=pltpu.VMEM),
            pl.BlockSpec(memory_space=pltpu.VMEM),
        ],
        out_specs=pl.BlockSpec(memory_space=pl.ANY),
        scratch_shapes=[
            pltpu.VMEM((M, n_loc), jnp.float32),
            pltpu.VMEM((M, n_loc), jnp.bfloat16),
            pltpu.VMEM((P, TILE, LANES), jnp.bfloat16),
            pltpu.VMEM((OUT_CHUNKS, OUT_R, n_loc), jnp.bfloat16),
            pltpu.SemaphoreType.DMA((IN_CHUNKS,)),
            pltpu.SemaphoreType.DMA((OUT_CHUNKS,)),
            pltpu.SemaphoreType.DMA((P,)),
            pltpu.SemaphoreType.DMA((P,)),
        ],
        compiler_params=pltpu.CompilerParams(collective_id=0),
    )(x, gamma2d, beta2d)


# device time: 18886 ns/iter; 1.1787x vs baseline; 1.1787x over previous
import os

import jax
import jax.numpy as jnp
from jax import lax
from jax.experimental import pallas as pl
from jax.experimental.pallas import tpu as pltpu

P = 32
EPS = 1e-5
LANES = 128
M = 2048
ROWS = M // LANES
TILE = 2 * ROWS
IN_CHUNKS = 8
IN_R = M // IN_CHUNKS
PACK_R = IN_R // LANES
OUT_CHUNKS = 4
OUT_R = M // OUT_CHUNKS

_OFFSETS = sorted(range(1, P), key=lambda j: min(j, P - j))


def kernel(x, gamma, beta):
    m, n_loc = x.shape
    n_glob = float(n_loc * P)

    gamma2d = gamma.reshape(1, n_loc)
    beta2d = beta.reshape(1, n_loc)

    def body(x_hbm, g_ref, b_ref, o_hbm, xv_ref, gx_ref, comm_ref, obuf_ref,
             in_sems, out_sems, send_sems, recv_sems):
        my = lax.axis_index("i")

        barrier = pltpu.get_barrier_semaphore()
        for j in _OFFSETS:
            pl.semaphore_signal(
                barrier,
                inc=1,
                device_id=((my + j) % P,),
                device_id_type=pl.DeviceIdType.MESH,
            )

        in_dmas = []
        for c in range(IN_CHUNKS):
            dma = pltpu.make_async_copy(
                x_hbm.at[pl.ds(c * IN_R, IN_R), :],
                xv_ref.at[pl.ds(c * IN_R, IN_R), :],
                in_sems.at[c],
            )
            dma.start()
            in_dmas.append(dma)

        gb = g_ref[:, :].astype(jnp.bfloat16)
        for c in range(IN_CHUNKS):
            in_dmas[c].wait()
            xc = xv_ref[pl.ds(c * IN_R, IN_R), :]
            r0 = c * PACK_R
            comm_ref[0, r0:r0 + PACK_R, :] = (
                jnp.sum(xc, axis=1).reshape(PACK_R, LANES).astype(jnp.bfloat16)
            )
            comm_ref[0, ROWS + r0:ROWS + r0 + PACK_R, :] = (
                jnp.sum(xc * xc, axis=1)
                .reshape(PACK_R, LANES)
                .astype(jnp.bfloat16)
            )

        pl.semaphore_wait(barrier, P - 1)

        rdmas = []
        if not os.environ.get("ABLATE_COMM"):
            for j in _OFFSETS:
                slot = P - j
                rdma = pltpu.make_async_remote_copy(
                    src_ref=comm_ref.at[0],
                    dst_ref=comm_ref.at[slot],
                    send_sem=send_sems.at[j],
                    recv_sem=recv_sems.at[slot],
                    device_id=((my + j) % P,),
                    device_id_type=pl.DeviceIdType.MESH,
                )
                rdma.start()
                rdmas.append(rdma)

        gx_ref[:, :] = gb * xv_ref[:, :].astype(jnp.bfloat16)

        for rdma in rdmas:
            rdma.wait()

        total = jnp.sum(comm_ref[:, :, :].astype(jnp.float32), axis=0)

        row_ids = lax.broadcasted_iota(jnp.int32, (m, ROWS), 0)
        sel = (lax.broadcasted_iota(jnp.int32, (m, ROWS), 1)
               == row_ids // LANES).astype(jnp.float32)
        lane_mask = (
            lax.broadcasted_iota(jnp.int32, (m, LANES), 1)
            == lax.broadcasted_iota(jnp.int32, (m, LANES), 0) % LANES
        )

        def unpack(packed):
            spread = jnp.dot(sel, packed, preferred_element_type=jnp.float32)
            return jnp.sum(
                jnp.where(lane_mask, spread, 0.0), axis=1, keepdims=True
            )

        mean = unpack(total[0:ROWS, :]) / n_glob
        var = unpack(total[ROWS:, :]) / n_glob - mean * mean
        rstd = lax.rsqrt(var + EPS)

        r_col = rstd.astype(jnp.bfloat16)
        mr_col = (mean * rstd).astype(jnp.bfloat16)
        bb = b_ref[:, :].astype(jnp.bfloat16)
        out_dmas = []
        for c in range(OUT_CHUNKS):
            rows = pl.ds(c * OUT_R, OUT_R)
            obuf_ref[c, :, :] = (
                gx_ref[rows, :] * r_col[c * OUT_R:(c + 1) * OUT_R, :]
                + (bb - gb * mr_col[c * OUT_R:(c + 1) * OUT_R, :])
            )
            dma = pltpu.make_async_copy(
                obuf_ref.at[c], o_hbm.at[rows, :], out_sems.at[c]
            )
            dma.start()
            out_dmas.append(dma)
        for dma in out_dmas:
            dma.wait()

    return pl.pallas_call(
        body,
        out_shape=jax.ShapeDtypeStruct((m, n_loc), jnp.bfloat16),
        in_specs=[
            pl.BlockSpec(memory_space=pl.ANY),
            pl.BlockSpec(memory_space=pltpu.VMEM),
            pl.BlockSpec(memory_space=pltpu.VMEM),
        ],
        out_specs=pl.BlockSpec(memory_space=pl.ANY),
        scratch_shapes=[
            pltpu.VMEM((M, n_loc), jnp.float32),
            pltpu.VMEM((M, n_loc), jnp.bfloat16),
            pltpu.VMEM((P, TILE, LANES), jnp.bfloat16),
            pltpu.VMEM((OUT_CHUNKS, OUT_R, n_loc), jnp.bfloat16),
            pltpu.SemaphoreType.DMA((IN_CHUNKS,)),
            pltpu.SemaphoreType.DMA((OUT_CHUNKS,)),
            pltpu.SemaphoreType.DMA((P,)),
            pltpu.SemaphoreType.DMA((P,)),
        ],
        compiler_params=pltpu.CompilerParams(collective_id=0),
    )(x, gamma2d, beta2d)
